# Initial kernel scaffold; baseline (speedup 1.0000x reference)
import jax
import jax.numpy as jnp
from jax.experimental import pallas as pl


def kernel(x, edge_index, batch, Wl, Wr, bn1_gamma, bn1_beta, dense_W, dense_b, bn2_gamma, bn2_beta):
    raise NotImplementedError("write your pallas kernel here")


# speedup vs baseline: 2.4305x; 2.4305x over previous
import functools

import jax
import jax.numpy as jnp
from jax import lax
from jax.experimental import pallas as pl
from jax.experimental.pallas import tpu as pltpu
from jax.experimental.pallas import tpu_sc as plsc

_N = 10000
_E = 160000
_D_IN = 256
_D_OUT = 64
_MAX_DEG = 10
_NBANK = _MAX_DEG + 1
_N_GRAPHS = 128
_EPS = 1e-5

_NC = 2
_NS = 16
_NW = _NC * _NS

_NP = 10112
_ROWS = 512
_N_PAD = 10240
_DH = 128

_EPT = 5120
_E_PAD = _NW * _EPT
_ECH2 = (_E_PAD // _NS) // 128

_mesh = functools.partial(plsc.VectorSubcoreMesh,
                          core_axis_name="c", subcore_axis_name="s")
_SC_PARAMS = pltpu.CompilerParams(needs_layout_passes=False)
_HI = lax.Precision.HIGHEST


def _wid():
    return lax.axis_index("s") * _NC + lax.axis_index("c")


def _counts_body(dst_hbm, zeros_hbm, cnt_out, cnt_v, dst_v):
    wid = _wid()
    pltpu.sync_copy(zeros_hbm, cnt_v)
    pltpu.sync_copy(dst_hbm.at[pl.ds(wid * _EPT, _EPT)], dst_v)
    ones = jnp.full((16,), 1, jnp.int32)

    def body(i, carry):
        d = dst_v[pl.ds(i * 16, 16)]
        plsc.addupdate_scatter(cnt_v, [d], ones)
        return carry

    lax.fori_loop(0, _EPT // 16, body, 0)
    pltpu.sync_copy(cnt_v, cnt_out.at[wid])


def _sc_counts(dst_pad, zeros_np_i32):
    return pl.kernel(
        _counts_body,
        out_type=jax.ShapeDtypeStruct((_NW, _NP), jnp.int32),
        mesh=_mesh(),
        scratch_types=[
            pltpu.VMEM((_NP,), jnp.int32),
            pltpu.VMEM((_EPT,), jnp.int32),
        ],
        compiler_params=_SC_PARAMS,
    )(dst_pad, zeros_np_i32)


def _deg_body(cnt_ref, deg_ref):
    c = cnt_ref[...]
    s = jnp.sum(c, axis=0, keepdims=True)
    deg_ref[...] = jnp.minimum(s, _MAX_DEG)


def _deg_reduce(counts):
    return pl.pallas_call(
        _deg_body,
        grid=(1,),
        in_specs=[pl.BlockSpec((_NW, _NP), lambda i: (0, 0))],
        out_specs=pl.BlockSpec((1, _NP), lambda i: (0, 0)),
        out_shape=jax.ShapeDtypeStruct((1, _NP), jnp.int32),
    )(counts)


def _hsum_body(table_hbm, src2_hbm, dst2_hbm, zeros_hbm,
               acc_out, src_v, dst_v, idxrow, buf, acc_sh, gsem):
    c = lax.axis_index("c")
    s = lax.axis_index("s")
    pltpu.sync_copy(src2_hbm.at[s], src_v)
    pltpu.sync_copy(dst2_hbm.at[s], dst_v)
    rows_per_tile = _NP // _NS
    pltpu.sync_copy(zeros_hbm, acc_sh.at[pl.ds(s * rows_per_tile, rows_per_tile)])
    plsc.subcore_barrier()
    base = c * _N

    def body(j, carry):
        for k in range(8):
            s16 = src_v[j, pl.ds(k * 16, 16)]
            idxrow[0, pl.ds(k * 16, 16)] = s16 + base
        pltpu.async_copy(table_hbm.at[idxrow.at[0]], buf, gsem).wait()
        pltpu.sync_copy(buf, acc_sh.at[dst_v.at[j]], add=True)
        return carry

    lax.fori_loop(0, _ECH2, body, 0)
    plsc.subcore_barrier()
    pltpu.sync_copy(acc_sh.at[pl.ds(s * rows_per_tile, rows_per_tile)],
                    acc_out.at[c, pl.ds(s * rows_per_tile, rows_per_tile)])


def _sc_hsum(table, src2, dst2, zeros_rows):
    return pl.kernel(
        _hsum_body,
        out_type=jax.ShapeDtypeStruct((_NC, _NP, _DH), jnp.float32),
        mesh=_mesh(),
        scratch_types=[
            pltpu.VMEM((_ECH2, 128), jnp.int32),
            pltpu.VMEM((_ECH2, 128), jnp.int32),
            pltpu.VMEM((1, 128), jnp.int32),
            pltpu.VMEM((128, _DH), jnp.float32),
            pltpu.VMEM_SHARED((_NP, _DH), jnp.float32),
            pltpu.SemaphoreType.DMA,
        ],
        compiler_params=_SC_PARAMS,
    )(table, src2, dst2, zeros_rows)


def _mfconv_body(x_ref, h_ref, deg_ref, wl_ref, wr_ref, out_ref):
    x = x_ref[...]
    h = h_ref[...]
    deg = deg_ref[...]
    yl = jax.lax.dot_general(h, wl_ref[...], (((1,), (1,)), ((), ())),
                             preferred_element_type=jnp.float32)
    yr = jax.lax.dot_general(x, wr_ref[...], (((1,), (1,)), ((), ())),
                             preferred_element_type=jnp.float32)
    y = yl + yr
    acc = jnp.zeros((x.shape[0], _D_OUT), jnp.float32)
    for i in range(_NBANK):
        acc = jnp.where(deg == i, y[:, i * _D_OUT:(i + 1) * _D_OUT], acc)
    out_ref[...] = jnp.maximum(acc, 0.0)


def _mfconv(x_pad, h_pad, deg_pad, wl_flat, wr_flat):
    return pl.pallas_call(
        _mfconv_body,
        grid=(_N_PAD // _ROWS,),
        in_specs=[
            pl.BlockSpec((_ROWS, _D_IN), lambda i: (i, 0)),
            pl.BlockSpec((_ROWS, _D_IN), lambda i: (i, 0)),
            pl.BlockSpec((_ROWS, 1), lambda i: (i, 0)),
            pl.BlockSpec((_NBANK * _D_OUT, _D_IN), lambda i: (0, 0)),
            pl.BlockSpec((_NBANK * _D_OUT, _D_IN), lambda i: (0, 0)),
        ],
        out_specs=pl.BlockSpec((_ROWS, _D_OUT), lambda i: (i, 0)),
        out_shape=jax.ShapeDtypeStruct((_N_PAD, _D_OUT), jnp.float32),
    )(x_pad, h_pad, deg_pad, wl_flat, wr_flat)


def _bn1_body(pre_ref, g_ref, b_ref, out_ref):
    a = pre_ref[...]
    valid = lax.broadcasted_iota(jnp.int32, (_NP, _D_OUT), 0) < _N
    a = jnp.where(valid, a, 0.0)
    mean = jnp.sum(a, axis=0, keepdims=True) / _N
    var = jnp.sum(a * a, axis=0, keepdims=True) / _N - mean * mean
    scale = g_ref[...] * lax.rsqrt(var + _EPS)
    shift = b_ref[...] - mean * scale
    abn = a * scale + shift
    abn = jnp.where(valid, abn, 0.0)
    io = lax.broadcasted_iota(jnp.int32, (_D_OUT, _D_OUT), 0)
    jo = lax.broadcasted_iota(jnp.int32, (_D_OUT, _D_OUT), 1)
    eye = (io == jo).astype(jnp.float32)
    out_ref[...] = jax.lax.dot_general(eye, abn, (((1,), (1,)), ((), ())),
                                       preferred_element_type=jnp.float32,
                                       precision=_HI)


def _bn1_t(pre, g1, b1):
    return pl.pallas_call(
        _bn1_body,
        grid=(1,),
        in_specs=[
            pl.BlockSpec((_NP, _D_OUT), lambda i: (0, 0)),
            pl.BlockSpec((1, _D_OUT), lambda i: (0, 0)),
            pl.BlockSpec((1, _D_OUT), lambda i: (0, 0)),
        ],
        out_specs=pl.BlockSpec((_D_OUT, _NP), lambda i: (0, 0)),
        out_shape=jax.ShapeDtypeStruct((_D_OUT, _NP), jnp.float32),
    )(pre, g1, b1)


_CS3 = 8192


def _maxpool_body(obnt_hbm, src_hbm, dst_hbm, pooled_out,
                  feat_v, acc_v, src_c, dst_c):
    wid = _wid()
    pltpu.sync_copy(obnt_hbm.at[wid], feat_v)
    pltpu.sync_copy(obnt_hbm.at[wid], acc_v)

    def chunk(ci, carry):
        pltpu.sync_copy(src_hbm.at[pl.ds(ci * _CS3, _CS3)], src_c)
        pltpu.sync_copy(dst_hbm.at[pl.ds(ci * _CS3, _CS3)], dst_c)

        def body(i, c2):
            s16 = src_c[pl.ds(i * 16, 16)]
            d16 = dst_c[pl.ds(i * 16, 16)]
            for f in range(2):
                s16f = s16 + (f * _NP)
                d16f = d16 + (f * _NP)
                vals = plsc.load_gather(feat_v, [s16f])
                cur = plsc.load_gather(acc_v, [d16f])
                new = jnp.maximum(cur, vals)
                plsc.store_scatter(acc_v, [d16f], new)
                chk = plsc.load_gather(acc_v, [d16f])
                lost = chk < new

                def cond(m):
                    return jnp.max(m.astype(jnp.int32)) > 0

                def retry(m):
                    plsc.store_scatter(acc_v, [d16f], new, mask=m)
                    cc = plsc.load_gather(acc_v, [d16f])
                    return jnp.logical_and(m, cc < new)

                lax.while_loop(cond, retry, lost)
            return c2

        lax.fori_loop(0, _CS3 // 16, body, 0)
        return carry

    lax.fori_loop(0, _E_PAD // _CS3, chunk, 0)
    pltpu.sync_copy(acc_v, pooled_out.at[wid])


def _sc_maxpool(obn_t2, src_pad, dst_pad):
    return pl.kernel(
        _maxpool_body,
        out_type=jax.ShapeDtypeStruct((_NW, 2 * _NP), jnp.float32),
        mesh=_mesh(),
        scratch_types=[
            pltpu.VMEM((2 * _NP,), jnp.float32),
            pltpu.VMEM((2 * _NP,), jnp.float32),
            pltpu.VMEM((_CS3,), jnp.int32),
            pltpu.VMEM((_CS3,), jnp.int32),
        ],
        compiler_params=_SC_PARAMS,
    )(obn_t2, src_pad, dst_pad)


def _final_body(pooled_ref, w_ref, b_ref, g2_ref, b2_ref, batch_ref, out_ref):
    pooled_t = pooled_ref[...]
    z = jax.lax.dot_general(w_ref[...], pooled_t, (((1,), (0,)), ((), ())),
                            preferred_element_type=jnp.float32)
    z = z + b_ref[...]
    z = jnp.maximum(z, 0.0)
    valid = lax.broadcasted_iota(jnp.int32, (_D_OUT, _NP), 1) < _N
    zv = jnp.where(valid, z, 0.0)
    mean = jnp.sum(zv, axis=1, keepdims=True) / _N
    var = jnp.sum(zv * zv, axis=1, keepdims=True) / _N - mean * mean
    scale = g2_ref[...] * lax.rsqrt(var + _EPS)
    shift = b2_ref[...] - mean * scale
    zbn = zv * scale + shift
    zbn = jnp.where(valid, zbn, 0.0)
    gid = lax.broadcasted_iota(jnp.int32, (_NP, _N_GRAPHS), 1)
    m = (batch_ref[...] == gid).astype(jnp.float32)
    gt = jax.lax.dot_general(zbn, m, (((1,), (0,)), ((), ())),
                             preferred_element_type=jnp.float32, precision=_HI)
    gt = jnp.tanh(gt)
    io = lax.broadcasted_iota(jnp.int32, (_N_GRAPHS, _N_GRAPHS), 0)
    jo = lax.broadcasted_iota(jnp.int32, (_N_GRAPHS, _N_GRAPHS), 1)
    eye = (io == jo).astype(jnp.float32)
    out_ref[...] = jax.lax.dot_general(eye, gt, (((1,), (1,)), ((), ())),
                                       preferred_element_type=jnp.float32,
                                       precision=_HI)


def _final(pooled_t, dense_W, dense_b2d, g2, b2, batch_col):
    return pl.pallas_call(
        _final_body,
        grid=(1,),
        in_specs=[
            pl.BlockSpec((_D_OUT, _NP), lambda i: (0, 0)),
            pl.BlockSpec((_D_OUT, _D_OUT), lambda i: (0, 0)),
            pl.BlockSpec((_D_OUT, 1), lambda i: (0, 0)),
            pl.BlockSpec((_D_OUT, 1), lambda i: (0, 0)),
            pl.BlockSpec((_D_OUT, 1), lambda i: (0, 0)),
            pl.BlockSpec((_NP, 1), lambda i: (0, 0)),
        ],
        out_specs=pl.BlockSpec((_N_GRAPHS, _D_OUT), lambda i: (0, 0)),
        out_shape=jax.ShapeDtypeStruct((_N_GRAPHS, _D_OUT), jnp.float32),
    )(pooled_t, dense_W, dense_b2d, g2, b2, batch_col)


def kernel(x, edge_index, batch, Wl, Wr, bn1_gamma, bn1_beta, dense_W,
           dense_b, bn2_gamma, bn2_beta):
    src = edge_index[0]
    dst = edge_index[1]
    i32 = jnp.int32

    pad_e = _E_PAD - _E
    arange_p = jnp.arange(pad_e, dtype=i32)
    dummy_dst = _N + (arange_p % 16)
    dst_a = jnp.concatenate([dst, dummy_dst])
    src_a3 = jnp.concatenate([src, dummy_dst])
    src_a2 = jnp.concatenate([src, arange_p % 64])

    zeros_np_i32 = jnp.zeros((_NP,), i32)
    counts = _sc_counts(dst_a, zeros_np_i32)
    degc = _deg_reduce(counts)

    table = jnp.concatenate([x[:, :_DH], x[:, _DH:]], axis=0)
    src_s2 = src_a2.reshape(_NS, _ECH2, 128)
    dst_s2 = dst_a.reshape(_NS, _ECH2, 128)
    zeros_rows = jnp.zeros((_NP // _NS, _DH), jnp.float32)
    hacc = _sc_hsum(table, src_s2, dst_s2, zeros_rows)
    h = jnp.concatenate([hacc[0], hacc[1]], axis=1)

    x_pad = jnp.pad(x, ((0, _N_PAD - _N), (0, 0)))
    h_pad = jnp.pad(h, ((0, _N_PAD - _NP), (0, 0)))
    deg_pad = jnp.pad(degc.reshape(_NP), (0, _N_PAD - _NP))[:, None]
    wl_flat = Wl.reshape(_NBANK * _D_OUT, _D_IN)
    wr_flat = Wr.reshape(_NBANK * _D_OUT, _D_IN)
    pre = _mfconv(x_pad, h_pad, deg_pad, wl_flat, wr_flat)

    obn_t = _bn1_t(pre[:_NP], bn1_gamma[None, :], bn1_beta[None, :])
    pooled_t = _sc_maxpool(obn_t.reshape(_NW, 2 * _NP),
                           src_a3, dst_a).reshape(_D_OUT, _NP)

    batch_col = jnp.pad(batch, (0, _NP - _N),
                        constant_values=_N_GRAPHS)[:, None].astype(i32)
    return _final(pooled_t, dense_W, dense_b[:, None],
                  bn2_gamma[:, None], bn2_beta[:, None], batch_col)


# speedup vs baseline: 4.2888x; 1.7646x over previous
import functools

import jax
import jax.numpy as jnp
from jax import lax
from jax.experimental import pallas as pl
from jax.experimental.pallas import tpu as pltpu
from jax.experimental.pallas import tpu_sc as plsc

_N = 10000
_E = 160000
_D_IN = 256
_D_OUT = 64
_MAX_DEG = 10
_NBANK = _MAX_DEG + 1
_N_GRAPHS = 128
_EPS = 1e-5

_NC = 2
_NS = 16
_NW = _NC * _NS

_NP = 10112
_ROWS = 512
_N_PAD = 10240
_DH = 128

_EPT = 5120
_E_PAD = _NW * _EPT
_CW2 = 64
_ECH2 = (_E_PAD // _NS) // _CW2

_mesh = functools.partial(plsc.VectorSubcoreMesh,
                          core_axis_name="c", subcore_axis_name="s")
_SC_PARAMS = pltpu.CompilerParams(needs_layout_passes=False)
_HI = lax.Precision.HIGHEST


def _wid():
    return lax.axis_index("s") * _NC + lax.axis_index("c")


def _counts_body(dst_hbm, zeros_hbm, cnt_out, cnt_v, dst_v):
    wid = _wid()
    pltpu.sync_copy(zeros_hbm, cnt_v)
    pltpu.sync_copy(dst_hbm.at[pl.ds(wid * _EPT, _EPT)], dst_v)
    ones = jnp.full((16,), 1, jnp.int32)

    def body(i, carry):
        d = dst_v[pl.ds(i * 16, 16)]
        plsc.addupdate_scatter(cnt_v, [d], ones)
        return carry

    lax.fori_loop(0, _EPT // 16, body, 0)
    pltpu.sync_copy(cnt_v, cnt_out.at[wid])


def _sc_counts(dst_pad, zeros_np_i32):
    return pl.kernel(
        _counts_body,
        out_type=jax.ShapeDtypeStruct((_NW, _NP), jnp.int32),
        mesh=_mesh(),
        scratch_types=[
            pltpu.VMEM((_NP,), jnp.int32),
            pltpu.VMEM((_EPT,), jnp.int32),
        ],
        compiler_params=_SC_PARAMS,
    )(dst_pad, zeros_np_i32)


def _deg_body(cnt_ref, deg_ref):
    c = cnt_ref[...]
    s = jnp.sum(c, axis=0, keepdims=True)
    deg_ref[...] = jnp.minimum(s, _MAX_DEG)


def _deg_reduce(counts):
    return pl.pallas_call(
        _deg_body,
        grid=(1,),
        in_specs=[pl.BlockSpec((_NW, _NP), lambda i: (0, 0))],
        out_specs=pl.BlockSpec((1, _NP), lambda i: (0, 0)),
        out_shape=jax.ShapeDtypeStruct((1, _NP), jnp.int32),
    )(counts)


def _hsum_body(table_hbm, src2_hbm, dst2_hbm, zeros_hbm,
               acc_out, src_v, dst_v, idxr0, idxr1, buf0, buf1, acc_sh,
               sem0, sem1):
    c = lax.axis_index("c")
    s = lax.axis_index("s")
    pltpu.sync_copy(src2_hbm.at[s], src_v)
    pltpu.sync_copy(dst2_hbm.at[s], dst_v)
    rows_per_tile = _NP // _NS
    pltpu.sync_copy(zeros_hbm, acc_sh.at[pl.ds(s * rows_per_tile, rows_per_tile)])
    plsc.subcore_barrier()
    base = c * _N

    def cidx(j, idxr):
        for k in range(_CW2 // 16):
            s16 = src_v[pl.ds(j * _CW2 + k * 16, 16)]
            idxr[0, pl.ds(k * 16, 16)] = s16 + base

    cidx(0, idxr0)
    pltpu.async_copy(table_hbm.at[idxr0.at[0]], buf0, sem0)

    def body(jj, carry):
        j0 = 2 * jj
        cidx(j0 + 1, idxr1)
        pltpu.async_copy(table_hbm.at[idxr1.at[0]], buf1, sem1)
        pltpu.make_async_copy(table_hbm.at[idxr0.at[0]], buf0, sem0).wait()
        pltpu.sync_copy(buf0, acc_sh.at[dst_v.at[j0]], add=True)
        cidx(j0 + 2, idxr0)
        pltpu.async_copy(table_hbm.at[idxr0.at[0]], buf0, sem0)
        pltpu.make_async_copy(table_hbm.at[idxr1.at[0]], buf1, sem1).wait()
        pltpu.sync_copy(buf1, acc_sh.at[dst_v.at[j0 + 1]], add=True)
        return carry

    lax.fori_loop(0, _ECH2 // 2 - 1, body, 0)
    jl = _ECH2 - 2
    cidx(jl + 1, idxr1)
    pltpu.async_copy(table_hbm.at[idxr1.at[0]], buf1, sem1)
    pltpu.make_async_copy(table_hbm.at[idxr0.at[0]], buf0, sem0).wait()
    pltpu.sync_copy(buf0, acc_sh.at[dst_v.at[jl]], add=True)
    pltpu.make_async_copy(table_hbm.at[idxr1.at[0]], buf1, sem1).wait()
    pltpu.sync_copy(buf1, acc_sh.at[dst_v.at[jl + 1]], add=True)
    plsc.subcore_barrier()
    pltpu.sync_copy(acc_sh.at[pl.ds(s * rows_per_tile, rows_per_tile)],
                    acc_out.at[c, pl.ds(s * rows_per_tile, rows_per_tile)])


def _sc_hsum(table, src2, dst2, zeros_rows):
    return pl.kernel(
        _hsum_body,
        out_type=jax.ShapeDtypeStruct((_NC, _NP, _DH), jnp.float32),
        mesh=_mesh(),
        scratch_types=[
            pltpu.VMEM((_E_PAD // _NS,), jnp.int32),
            pltpu.VMEM((_ECH2, _CW2), jnp.int32),
            pltpu.VMEM((1, _CW2), jnp.int32),
            pltpu.VMEM((1, _CW2), jnp.int32),
            pltpu.VMEM((_CW2, _DH), jnp.float32),
            pltpu.VMEM((_CW2, _DH), jnp.float32),
            pltpu.VMEM_SHARED((_NP, _DH), jnp.float32),
            pltpu.SemaphoreType.DMA,
            pltpu.SemaphoreType.DMA,
        ],
        compiler_params=_SC_PARAMS,
    )(table, src2, dst2, zeros_rows)


def _mfconv_body(x_ref, h_ref, deg_ref, wl_ref, wr_ref, out_ref):
    x = x_ref[...]
    h = h_ref[...]
    deg = deg_ref[...]
    yl = jax.lax.dot_general(h, wl_ref[...], (((1,), (1,)), ((), ())),
                             preferred_element_type=jnp.float32)
    yr = jax.lax.dot_general(x, wr_ref[...], (((1,), (1,)), ((), ())),
                             preferred_element_type=jnp.float32)
    y = yl + yr
    acc = jnp.zeros((x.shape[0], _D_OUT), jnp.float32)
    for i in range(_NBANK):
        acc = jnp.where(deg == i, y[:, i * _D_OUT:(i + 1) * _D_OUT], acc)
    out_ref[...] = jnp.maximum(acc, 0.0)


def _mfconv(x_pad, h_pad, deg_pad, wl_flat, wr_flat):
    return pl.pallas_call(
        _mfconv_body,
        grid=(_N_PAD // _ROWS,),
        in_specs=[
            pl.BlockSpec((_ROWS, _D_IN), lambda i: (i, 0)),
            pl.BlockSpec((_ROWS, _D_IN), lambda i: (i, 0)),
            pl.BlockSpec((_ROWS, 1), lambda i: (i, 0)),
            pl.BlockSpec((_NBANK * _D_OUT, _D_IN), lambda i: (0, 0)),
            pl.BlockSpec((_NBANK * _D_OUT, _D_IN), lambda i: (0, 0)),
        ],
        out_specs=pl.BlockSpec((_ROWS, _D_OUT), lambda i: (i, 0)),
        out_shape=jax.ShapeDtypeStruct((_N_PAD, _D_OUT), jnp.float32),
    )(x_pad, h_pad, deg_pad, wl_flat, wr_flat)


def _bn1_body(pre_ref, g_ref, b_ref, out_ref):
    a = pre_ref[...]
    valid = lax.broadcasted_iota(jnp.int32, (_NP, _D_OUT), 0) < _N
    a = jnp.where(valid, a, 0.0)
    mean = jnp.sum(a, axis=0, keepdims=True) / _N
    var = jnp.sum(a * a, axis=0, keepdims=True) / _N - mean * mean
    scale = g_ref[...] * lax.rsqrt(var + _EPS)
    shift = b_ref[...] - mean * scale
    abn = a * scale + shift
    abn = jnp.where(valid, abn, 0.0)
    io = lax.broadcasted_iota(jnp.int32, (_D_OUT, _D_OUT), 0)
    jo = lax.broadcasted_iota(jnp.int32, (_D_OUT, _D_OUT), 1)
    eye = (io == jo).astype(jnp.float32)
    out_ref[...] = jax.lax.dot_general(eye, abn, (((1,), (1,)), ((), ())),
                                       preferred_element_type=jnp.float32,
                                       precision=_HI)


def _bn1_t(pre, g1, b1):
    return pl.pallas_call(
        _bn1_body,
        grid=(1,),
        in_specs=[
            pl.BlockSpec((_NP, _D_OUT), lambda i: (0, 0)),
            pl.BlockSpec((1, _D_OUT), lambda i: (0, 0)),
            pl.BlockSpec((1, _D_OUT), lambda i: (0, 0)),
        ],
        out_specs=pl.BlockSpec((_D_OUT, _NP), lambda i: (0, 0)),
        out_shape=jax.ShapeDtypeStruct((_D_OUT, _NP), jnp.float32),
    )(pre, g1, b1)


_CS3 = 4096
_EHALF = _E_PAD // 2
_NF3 = 4


def _maxpool_body(obnt_hbm, src_hbm, dst_hbm, pooled_out,
                  feat_v, acc_v, src_c, dst_c):
    c = lax.axis_index("c")
    s = lax.axis_index("s")
    wid = s * _NC + c
    pltpu.sync_copy(obnt_hbm.at[s], feat_v)
    pltpu.sync_copy(obnt_hbm.at[s], acc_v)
    ebase = c * _EHALF

    def chunk(ci, carry):
        pltpu.sync_copy(src_hbm.at[pl.ds(ebase + ci * _CS3, _CS3)], src_c)
        pltpu.sync_copy(dst_hbm.at[pl.ds(ebase + ci * _CS3, _CS3)], dst_c)

        def body(i, c2):
            s16 = src_c[pl.ds(i * 16, 16)]
            d16 = dst_c[pl.ds(i * 16, 16)]
            news = []
            dfs = []
            lost_any = None
            for f in range(_NF3):
                s16f = s16 + (f * _NP)
                d16f = d16 + (f * _NP)
                vals = plsc.load_gather(feat_v, [s16f])
                cur = plsc.load_gather(acc_v, [d16f])
                new = jnp.maximum(cur, vals)
                plsc.store_scatter(acc_v, [d16f], new)
                news.append(new)
                dfs.append(d16f)
            lost = []
            for f in range(_NF3):
                chk = plsc.load_gather(acc_v, [dfs[f]])
                lf = chk < news[f]
                lost.append(lf)
                lost_any = lf if lost_any is None else jnp.logical_or(lost_any, lf)
            npc = plsc.all_reduce_population_count(lost_any)

            @pl.when(npc[0] > 0)
            def _():
                for f in range(_NF3):
                    def cond(m):
                        return jnp.max(m.astype(jnp.int32)) > 0

                    def retry(m, f=f):
                        plsc.store_scatter(acc_v, [dfs[f]], news[f], mask=m)
                        cc = plsc.load_gather(acc_v, [dfs[f]])
                        return jnp.logical_and(m, cc < news[f])

                    lax.while_loop(cond, retry, lost[f])
            return c2

        lax.fori_loop(0, _CS3 // 16, body, 0)
        return carry

    lax.fori_loop(0, _EHALF // _CS3, chunk, 0)
    pltpu.sync_copy(acc_v, pooled_out.at[c, pl.ds(s * (_NF3 * _NP), _NF3 * _NP)])


def _sc_maxpool(obn_t4, src_pad, dst_pad):
    return pl.kernel(
        _maxpool_body,
        out_type=jax.ShapeDtypeStruct((_NC, _NS * _NF3 * _NP), jnp.float32),
        mesh=_mesh(),
        scratch_types=[
            pltpu.VMEM((_NF3 * _NP,), jnp.float32),
            pltpu.VMEM((_NF3 * _NP,), jnp.float32),
            pltpu.VMEM((_CS3,), jnp.int32),
            pltpu.VMEM((_CS3,), jnp.int32),
        ],
        compiler_params=_SC_PARAMS,
    )(obn_t4, src_pad, dst_pad)


def _final_body(pooled_ref, w_ref, b_ref, g2_ref, b2_ref, batch_ref, out_ref):
    pooled_t = jnp.maximum(pooled_ref[0], pooled_ref[1])
    z = jax.lax.dot_general(w_ref[...], pooled_t, (((1,), (0,)), ((), ())),
                            preferred_element_type=jnp.float32)
    z = z + b_ref[...]
    z = jnp.maximum(z, 0.0)
    valid = lax.broadcasted_iota(jnp.int32, (_D_OUT, _NP), 1) < _N
    zv = jnp.where(valid, z, 0.0)
    mean = jnp.sum(zv, axis=1, keepdims=True) / _N
    var = jnp.sum(zv * zv, axis=1, keepdims=True) / _N - mean * mean
    scale = g2_ref[...] * lax.rsqrt(var + _EPS)
    shift = b2_ref[...] - mean * scale
    zbn = zv * scale + shift
    zbn = jnp.where(valid, zbn, 0.0)
    gid = lax.broadcasted_iota(jnp.int32, (_NP, _N_GRAPHS), 1)
    m = (batch_ref[...] == gid).astype(jnp.float32)
    gt = jax.lax.dot_general(zbn, m, (((1,), (0,)), ((), ())),
                             preferred_element_type=jnp.float32, precision=_HI)
    gt = jnp.tanh(gt)
    io = lax.broadcasted_iota(jnp.int32, (_N_GRAPHS, _N_GRAPHS), 0)
    jo = lax.broadcasted_iota(jnp.int32, (_N_GRAPHS, _N_GRAPHS), 1)
    eye = (io == jo).astype(jnp.float32)
    out_ref[...] = jax.lax.dot_general(eye, gt, (((1,), (1,)), ((), ())),
                                       preferred_element_type=jnp.float32,
                                       precision=_HI)


def _final(pooled_t, dense_W, dense_b2d, g2, b2, batch_col):
    return pl.pallas_call(
        _final_body,
        grid=(1,),
        in_specs=[
            pl.BlockSpec((_NC, _D_OUT, _NP), lambda i: (0, 0, 0)),
            pl.BlockSpec((_D_OUT, _D_OUT), lambda i: (0, 0)),
            pl.BlockSpec((_D_OUT, 1), lambda i: (0, 0)),
            pl.BlockSpec((_D_OUT, 1), lambda i: (0, 0)),
            pl.BlockSpec((_D_OUT, 1), lambda i: (0, 0)),
            pl.BlockSpec((_NP, 1), lambda i: (0, 0)),
        ],
        out_specs=pl.BlockSpec((_N_GRAPHS, _D_OUT), lambda i: (0, 0)),
        out_shape=jax.ShapeDtypeStruct((_N_GRAPHS, _D_OUT), jnp.float32),
    )(pooled_t, dense_W, dense_b2d, g2, b2, batch_col)


def kernel(x, edge_index, batch, Wl, Wr, bn1_gamma, bn1_beta, dense_W,
           dense_b, bn2_gamma, bn2_beta):
    src = edge_index[0]
    dst = edge_index[1]
    i32 = jnp.int32

    pad_e = _E_PAD - _E
    arange_p = jnp.arange(pad_e, dtype=i32)
    dummy_dst = _N + (arange_p % 16)
    dst_a = jnp.concatenate([dst, dummy_dst])
    src_a3 = jnp.concatenate([src, dummy_dst])
    src_a2 = jnp.concatenate([src, arange_p % 64])

    zeros_np_i32 = jnp.zeros((_NP,), i32)
    counts = _sc_counts(dst_a, zeros_np_i32)
    degc = _deg_reduce(counts)

    table = jnp.concatenate([x[:, :_DH], x[:, _DH:]], axis=0)
    src_s2 = src_a2.reshape(_NS, _E_PAD // _NS)
    dst_s2 = dst_a.reshape(_NS, _ECH2, _CW2)
    zeros_rows = jnp.zeros((_NP // _NS, _DH), jnp.float32)
    hacc = _sc_hsum(table, src_s2, dst_s2, zeros_rows)
    h = jnp.concatenate([hacc[0], hacc[1]], axis=1)

    x_pad = jnp.pad(x, ((0, _N_PAD - _N), (0, 0)))
    h_pad = jnp.pad(h, ((0, _N_PAD - _NP), (0, 0)))
    deg_pad = jnp.pad(degc.reshape(_NP), (0, _N_PAD - _NP))[:, None]
    wl_flat = Wl.reshape(_NBANK * _D_OUT, _D_IN)
    wr_flat = Wr.reshape(_NBANK * _D_OUT, _D_IN)
    pre = _mfconv(x_pad, h_pad, deg_pad, wl_flat, wr_flat)

    obn_t = _bn1_t(pre[:_NP], bn1_gamma[None, :], bn1_beta[None, :])
    praw = _sc_maxpool(obn_t.reshape(_NS, _NF3 * _NP), src_a3, dst_a)
    pooled_pair = praw.reshape(_NC, _D_OUT, _NP)

    batch_col = jnp.pad(batch, (0, _NP - _N),
                        constant_values=_N_GRAPHS)[:, None].astype(i32)
    return _final(pooled_pair, dense_W, dense_b[:, None],
                  bn2_gamma[:, None], bn2_beta[:, None], batch_col)
